# initial kernel scaffold (unmeasured)
import jax
import jax.numpy as jnp
from jax import lax
from jax.experimental import pallas as pl
from jax.experimental.pallas import tpu as pltpu

N_DEV = 4
N_TOK = 2048
D_MODEL = 512
D_HID = 1024
N_EXPERTS = 32
E_LOCAL = N_EXPERTS // N_DEV
CHUNK = N_TOK // N_DEV


def kernel(x, router_W, route_idx, expert_W, shared_W):
    def body(
        x_ref,
        rw_ref,
        idx_ref,
        ew_ref,
        sw_ref,
        out_ref,
        acc_ref,
        send_buf,
        recv_buf,
        send_sems,
        recv_sems,
    ):
        my = lax.axis_index("i")
        left = lax.rem(my + N_DEV - 1, N_DEV)
        right = lax.rem(my + 1, N_DEV)

        barrier_sem = pltpu.get_barrier_semaphore()
        for nbr in (left, right):
            pl.semaphore_signal(
                barrier_sem,
                inc=1,
                device_id=(nbr,),
                device_id_type=pl.DeviceIdType.MESH,
            )
        pl.semaphore_wait(barrier_sem, 2)

        x32 = x_ref[...]
        scores = jnp.dot(x32, rw_ref[...], preferred_element_type=jnp.float32)
        scores = scores - jnp.max(scores, axis=-1, keepdims=True)
        ex = jnp.exp(scores)
        probs = ex / jnp.sum(ex, axis=-1, keepdims=True)

        eids = lax.broadcasted_iota(jnp.int32, (N_TOK, N_EXPERTS), 1)
        routed = idx_ref[...]
        wsel = jnp.where(eids == routed, probs, 0.0)

        acc = jnp.zeros((N_TOK, D_HID), jnp.float32)
        for el in range(E_LOCAL):
            geid = my * E_LOCAL + el
            wcol = jnp.sum(
                jnp.where(eids == geid, wsel, 0.0), axis=1, keepdims=True
            )
            xw = (x32 * wcol).astype(jnp.bfloat16)
            acc = acc + jnp.dot(
                xw,
                ew_ref[el].astype(jnp.bfloat16),
                preferred_element_type=jnp.float32,
            )
        acc_ref[...] = acc

        for t in range(N_DEV - 1):
            c_send = lax.rem(my - t - 1 + 2 * N_DEV, N_DEV)
            chunk = acc_ref[pl.ds(c_send * CHUNK, CHUNK), :]
            if t > 0:
                chunk = chunk + recv_buf[t - 1].astype(jnp.float32)
            send_buf[t] = chunk.astype(jnp.bfloat16)
            rdma = pltpu.make_async_remote_copy(
                src_ref=send_buf.at[t],
                dst_ref=recv_buf.at[t],
                send_sem=send_sems.at[t],
                recv_sem=recv_sems.at[t],
                device_id=(right,),
                device_id_type=pl.DeviceIdType.MESH,
            )
            rdma.start()
            rdma.wait()

        x_mine = x_ref[pl.ds(my * CHUNK, CHUNK), :]
        shared = jnp.dot(
            x_mine.astype(jnp.bfloat16),
            sw_ref[...].astype(jnp.bfloat16),
            preferred_element_type=jnp.float32,
        )
        out_ref[...] = (
            recv_buf[N_DEV - 2].astype(jnp.float32)
            + acc_ref[pl.ds(my * CHUNK, CHUNK), :]
            + shared
        )

    out_shape = jax.ShapeDtypeStruct((CHUNK, D_HID), jnp.float32)
    return pl.pallas_call(
        body,
        out_shape=out_shape,
        in_specs=[pl.BlockSpec(memory_space=pltpu.VMEM)] * 5,
        out_specs=pl.BlockSpec(memory_space=pltpu.VMEM),
        scratch_shapes=[
            pltpu.VMEM((N_TOK, D_HID), jnp.float32),
            pltpu.VMEM((N_DEV - 1, CHUNK, D_HID), jnp.bfloat16),
            pltpu.VMEM((N_DEV - 1, CHUNK, D_HID), jnp.bfloat16),
            pltpu.SemaphoreType.DMA((N_DEV - 1,)),
            pltpu.SemaphoreType.DMA((N_DEV - 1,)),
        ],
        compiler_params=pltpu.CompilerParams(collective_id=0),
    )(x, router_W, route_idx, expert_W, shared_W)


# baseline (device time: 82115 ns/iter reference)
import jax
import jax.numpy as jnp
from jax import lax
from jax.experimental import pallas as pl
from jax.experimental.pallas import tpu as pltpu

N_DEV = 4
N_TOK = 2048
D_MODEL = 512
D_HID = 1024
N_EXPERTS = 32
E_LOCAL = N_EXPERTS // N_DEV
CHUNK = N_TOK // N_DEV


def kernel(x, router_W, route_idx, expert_W, shared_W):
    def body(
        x_ref,
        rw_ref,
        idx_ref,
        ew_ref,
        sw_ref,
        out_ref,
        acc_ref,
        send_buf,
        recv_buf,
        send_sems,
        recv_sems,
    ):
        my = lax.axis_index("i")
        left = lax.rem(my + N_DEV - 1, N_DEV)
        right = lax.rem(my + 1, N_DEV)

        barrier_sem = pltpu.get_barrier_semaphore()
        for nbr in (left, right):
            pl.semaphore_signal(
                barrier_sem,
                inc=1,
                device_id=(nbr,),
                device_id_type=pl.DeviceIdType.MESH,
            )
        pl.semaphore_wait(barrier_sem, 2)

        x32 = x_ref[...]
        scores = jnp.dot(x32, rw_ref[...], preferred_element_type=jnp.float32)
        scores = scores - jnp.max(scores, axis=-1, keepdims=True)
        ex = jnp.exp(scores)
        probs = ex / jnp.sum(ex, axis=-1, keepdims=True)

        eids = lax.broadcasted_iota(jnp.int32, (N_TOK, N_EXPERTS), 1)
        routed = idx_ref[...]
        wsel = jnp.where(eids == routed, probs, 0.0)

        acc = jnp.zeros((N_TOK, D_HID), jnp.float32)
        for el in range(E_LOCAL):
            geid = my * E_LOCAL + el
            wcol = jnp.sum(
                jnp.where(eids == geid, wsel, 0.0), axis=1, keepdims=True
            )
            xw = (x32 * wcol).astype(jnp.bfloat16)
            acc = acc + jnp.dot(
                xw,
                ew_ref[el].astype(jnp.bfloat16),
                preferred_element_type=jnp.float32,
            )
        acc_ref[...] = acc

        for t in range(N_DEV - 1):
            c_send = lax.rem(my - t - 1 + 2 * N_DEV, N_DEV)
            chunk = acc_ref[pl.ds(c_send * CHUNK, CHUNK), :]
            if t > 0:
                chunk = chunk + recv_buf[t - 1].astype(jnp.float32)
            send_buf[t] = chunk.astype(jnp.bfloat16)
            rdma = pltpu.make_async_remote_copy(
                src_ref=send_buf.at[t],
                dst_ref=recv_buf.at[t],
                send_sem=send_sems.at[t],
                recv_sem=recv_sems.at[t],
                device_id=(right,),
                device_id_type=pl.DeviceIdType.MESH,
            )
            rdma.start()
            rdma.wait()

        x_mine = x_ref[pl.ds(my * CHUNK, CHUNK), :]
        shared = jnp.dot(
            x_mine.astype(jnp.bfloat16),
            sw_ref[...].astype(jnp.bfloat16),
            preferred_element_type=jnp.float32,
        )
        out_ref[...] = (
            recv_buf[N_DEV - 2].astype(jnp.float32)
            + acc_ref[pl.ds(my * CHUNK, CHUNK), :]
            + shared
        )

    out_shape = jax.ShapeDtypeStruct((CHUNK, D_HID), jnp.float32)
    return pl.pallas_call(
        body,
        out_shape=out_shape,
        in_specs=[pl.BlockSpec(memory_space=pltpu.VMEM)] * 5,
        out_specs=pl.BlockSpec(memory_space=pltpu.VMEM),
        scratch_shapes=[
            pltpu.VMEM((N_TOK, D_HID), jnp.float32),
            pltpu.VMEM((N_DEV - 1, CHUNK, D_HID), jnp.bfloat16),
            pltpu.VMEM((N_DEV - 1, CHUNK, D_HID), jnp.bfloat16),
            pltpu.SemaphoreType.DMA((N_DEV - 1,)),
            pltpu.SemaphoreType.DMA((N_DEV - 1,)),
        ],
        compiler_params=pltpu.CompilerParams(
            collective_id=0, vmem_limit_bytes=100 * 1024 * 1024
        ),
    )(x, router_W, route_idx, expert_W, shared_W)


# device time: 64766 ns/iter; 1.2679x vs baseline; 1.2679x over previous
import jax
import jax.numpy as jnp
from jax import lax
from jax.experimental import pallas as pl
from jax.experimental.pallas import tpu as pltpu

N_DEV = 4
N_TOK = 2048
D_MODEL = 512
D_HID = 1024
N_EXPERTS = 32
E_LOCAL = N_EXPERTS // N_DEV
CHUNK = N_TOK // N_DEV


def kernel(x, router_W, route_idx, expert_W, shared_W):
    def body(
        x_ref,
        rw_ref,
        idx_ref,
        ew_ref,
        sw_ref,
        out_ref,
        send_buf,
        recv_buf,
        send_sems,
        recv_sems,
    ):
        my = lax.axis_index("i")
        left = lax.rem(my + N_DEV - 1, N_DEV)
        right = lax.rem(my + 1, N_DEV)

        barrier_sem = pltpu.get_barrier_semaphore()
        for nbr in (left, right):
            pl.semaphore_signal(
                barrier_sem,
                inc=1,
                device_id=(nbr,),
                device_id_type=pl.DeviceIdType.MESH,
            )
        pl.semaphore_wait(barrier_sem, 2)

        ew_bf = [ew_ref[el].astype(jnp.bfloat16) for el in range(E_LOCAL)]

        def partial_chunk(c):
            xs = x_ref[pl.ds(c * CHUNK, CHUNK), :]
            idx = idx_ref[pl.ds(c * CHUNK, CHUNK), :]
            scores = jnp.dot(
                xs, rw_ref[...], preferred_element_type=jnp.float32
            )
            scores = scores - jnp.max(scores, axis=-1, keepdims=True)
            exs = jnp.exp(scores)
            probs = exs / jnp.sum(exs, axis=-1, keepdims=True)
            eids = lax.broadcasted_iota(jnp.int32, (CHUNK, N_EXPERTS), 1)
            wsel = jnp.where(eids == idx, probs, 0.0)
            acc = jnp.zeros((CHUNK, D_HID), jnp.float32)
            for el in range(E_LOCAL):
                geid = my * E_LOCAL + el
                wcol = jnp.sum(
                    jnp.where(eids == geid, wsel, 0.0), axis=1, keepdims=True
                )
                acc = acc + jnp.dot(
                    (xs * wcol).astype(jnp.bfloat16),
                    ew_bf[el],
                    preferred_element_type=jnp.float32,
                )
            return acc

        rdmas = []
        for t in range(N_DEV - 1):
            c_send = lax.rem(my - t - 1 + 2 * N_DEV, N_DEV)
            chunk = partial_chunk(c_send)
            if t > 0:
                rdmas[t - 1].wait()
                chunk = chunk + recv_buf[t - 1].astype(jnp.float32)
            send_buf[t] = chunk.astype(jnp.bfloat16)
            rdma = pltpu.make_async_remote_copy(
                src_ref=send_buf.at[t],
                dst_ref=recv_buf.at[t],
                send_sem=send_sems.at[t],
                recv_sem=recv_sems.at[t],
                device_id=(right,),
                device_id_type=pl.DeviceIdType.MESH,
            )
            rdma.start()
            rdmas.append(rdma)

        p_mine = partial_chunk(my)
        x_mine = x_ref[pl.ds(my * CHUNK, CHUNK), :]
        shared = jnp.dot(
            x_mine.astype(jnp.bfloat16),
            sw_ref[...].astype(jnp.bfloat16),
            preferred_element_type=jnp.float32,
        )
        rdmas[N_DEV - 2].wait()
        out_ref[...] = recv_buf[N_DEV - 2].astype(jnp.float32) + p_mine + shared

    out_shape = jax.ShapeDtypeStruct((CHUNK, D_HID), jnp.float32)
    return pl.pallas_call(
        body,
        out_shape=out_shape,
        in_specs=[pl.BlockSpec(memory_space=pltpu.VMEM)] * 5,
        out_specs=pl.BlockSpec(memory_space=pltpu.VMEM),
        scratch_shapes=[
            pltpu.VMEM((N_DEV - 1, CHUNK, D_HID), jnp.bfloat16),
            pltpu.VMEM((N_DEV - 1, CHUNK, D_HID), jnp.bfloat16),
            pltpu.SemaphoreType.DMA((N_DEV - 1,)),
            pltpu.SemaphoreType.DMA((N_DEV - 1,)),
        ],
        compiler_params=pltpu.CompilerParams(
            collective_id=0, vmem_limit_bytes=100 * 1024 * 1024
        ),
    )(x, router_W, route_idx, expert_W, shared_W)
